# baseline (device time: 19046 ns/iter reference)
import jax
import jax.numpy as jnp
from jax import lax
from jax.experimental import pallas as pl
from jax.experimental.pallas import tpu as pltpu

N_DEV = 8
M_PER = 128
GELU_C = 0.7978845608028654

PMASK = {"x": 1, "y": 3, "z": 4}

BIT = {
    "x": lambda v: (v & 1) ^ ((v >> 1) & 1),
    "y": lambda v: (v >> 1) & 1,
    "z": lambda v: (v >> 2) & 1,
}


def _chunk_of(bits):
    return bits["z"] * 4 + bits["y"] * 2 + (bits["x"] ^ bits["y"])


def _gelu(y):
    return 0.5 * y * (1.0 + jnp.tanh(GELU_C * (y + 0.044715 * y * y * y)))


STREAMS = [
    (0, 256, ("z", "y", "x")),
    (256, 128, ("z", "y", "x")),
    (384, 256, ("y", "x", "z")),
    (640, 128, ("y", "x", "z")),
    (768, 128, ("x", "z", "y")),
    (896, 128, ("x", "z", "y")),
]
ORDER_SIS = [4, 5, 0, 1, 2, 3]

R1_COMBOS = [(1, 1), (1, 0), (0, 1), (0, 0)]
R2_COMBOS = [(1,), (0,)]
R1_BASE, R2_BASE, R3_SLOT = 0, 4, 6


def kernel(x, w_mat):
    m, k_per = x.shape
    _, n = w_mat.shape

    n_streams = len(STREAMS)

    def body(x_ref, w_ref, out_ref, p_ref, *rest):
        send_bufs = rest[0:2 * n_streams:2]
        recv_bufs = rest[1:2 * n_streams:2]
        send_sems, recv_sems = rest[2 * n_streams], rest[2 * n_streams + 1]

        my = lax.axis_index("i")
        mybit = {a: BIT[a](my) for a in "xyz"}

        barrier_sem = pltpu.get_barrier_semaphore()
        for mask in (1, 3, 4):
            pl.semaphore_signal(
                barrier_sem, inc=1,
                device_id=(my ^ mask,), device_id_type=pl.DeviceIdType.MESH,
            )
        pl.semaphore_wait(barrier_sem, 3)

        p_ref[:, :] = jnp.dot(
            x_ref[:, :], w_ref[:, :], preferred_element_type=jnp.float32
        ).astype(jnp.bfloat16)

        def send_chunk_bits(si, r, combo):
            _, _, order = STREAMS[si]
            bits = dict(mybit)
            bits[order[r]] = mybit[order[r]] ^ 1
            for j, f in enumerate(order[r + 1:]):
                bits[f] = mybit[f] ^ combo[j]
            return bits

        def rows(c):
            return pl.ds(c * M_PER, M_PER)

        def make_rdma(si, slot, a):
            sb, rb = send_bufs[si], recv_bufs[si]
            return pltpu.make_async_remote_copy(
                src_ref=sb.at[slot],
                dst_ref=rb.at[slot],
                send_sem=send_sems.at[si, slot],
                recv_sem=recv_sems.at[si, slot],
                device_id=(my ^ PMASK[a],),
                device_id_type=pl.DeviceIdType.MESH,
            )

        rdmas = {}

        def start_r1(si):
            c0, w, order = STREAMS[si]
            for k, combo in enumerate(R1_COMBOS):
                c = _chunk_of(send_chunk_bits(si, 0, combo))
                slot = R1_BASE + k
                rdma = pltpu.make_async_remote_copy(
                    src_ref=p_ref.at[rows(c), pl.ds(c0, w)],
                    dst_ref=recv_bufs[si].at[slot],
                    send_sem=send_sems.at[si, slot],
                    recv_sem=recv_sems.at[si, slot],
                    device_id=(my ^ PMASK[order[0]],),
                    device_id_type=pl.DeviceIdType.MESH,
                )
                rdmas[si, slot] = rdma
                rdma.start()

        def start_r2(si):
            c0, w, order = STREAMS[si]
            for k, combo in enumerate(R2_COMBOS):
                c = _chunk_of(send_chunk_bits(si, 1, combo))
                r1_slot = R1_BASE + (0 if combo[0] == 1 else 1)
                rdmas[si, r1_slot].wait_recv()
                slot = R2_BASE + k
                send_bufs[si][slot, :, :] = (
                    p_ref[rows(c), c0:c0 + w] + recv_bufs[si][r1_slot, :, :]
                )
                rdmas[si, slot] = make_rdma(si, slot, order[1])
                rdmas[si, slot].start()

        def start_r3(si):
            c0, w, order = STREAMS[si]
            rdmas[si, R1_BASE + 2].wait_recv()
            rdmas[si, R2_BASE + 0].wait_recv()
            c = _chunk_of(send_chunk_bits(si, 2, ()))
            send_bufs[si][R3_SLOT, :, :] = (
                p_ref[rows(c), c0:c0 + w]
                + recv_bufs[si][R1_BASE + 2, :, :]
                + recv_bufs[si][R2_BASE + 0, :, :]
            )
            rdmas[si, R3_SLOT] = make_rdma(si, R3_SLOT, order[2])
            rdmas[si, R3_SLOT].start()

        def finish(si):
            c0, w, _ = STREAMS[si]
            rdmas[si, R1_BASE + 3].wait_recv()
            rdmas[si, R2_BASE + 1].wait_recv()
            rdmas[si, R3_SLOT].wait_recv()
            acc = (
                p_ref[rows(my), c0:c0 + w].astype(jnp.float32)
                + recv_bufs[si][R1_BASE + 3, :, :].astype(jnp.float32)
                + recv_bufs[si][R2_BASE + 1, :, :].astype(jnp.float32)
                + recv_bufs[si][R3_SLOT, :, :].astype(jnp.float32)
            )
            out_ref[:, c0:c0 + w] = _gelu(acc)

        for si in ORDER_SIS:
            start_r1(si)
        for phase in (start_r2, start_r3, finish):
            for si in ORDER_SIS:
                phase(si)
        for si in ORDER_SIS:
            for slot in range(7):
                rdmas[si, slot].wait_send()

    scratch = [pltpu.VMEM((m, n), jnp.bfloat16)]
    for c0, w, order in STREAMS:
        scratch.append(pltpu.VMEM((7, M_PER, w), jnp.bfloat16))
        scratch.append(pltpu.VMEM((7, M_PER, w), jnp.bfloat16))
    scratch.append(pltpu.SemaphoreType.DMA((n_streams, 7)))
    scratch.append(pltpu.SemaphoreType.DMA((n_streams, 7)))

    return pl.pallas_call(
        body,
        out_shape=jax.ShapeDtypeStruct((M_PER, n), jnp.float32),
        in_specs=[
            pl.BlockSpec(memory_space=pltpu.VMEM),
            pl.BlockSpec(memory_space=pltpu.VMEM),
        ],
        out_specs=pl.BlockSpec(memory_space=pltpu.VMEM),
        scratch_shapes=scratch,
        compiler_params=pltpu.CompilerParams(collective_id=0),
    )(x, w_mat)


# device time: 18673 ns/iter; 1.0200x vs baseline; 1.0200x over previous
import jax
import jax.numpy as jnp
from jax import lax
from jax.experimental import pallas as pl
from jax.experimental.pallas import tpu as pltpu

N_DEV = 8
M_PER = 128
GELU_C = 0.7978845608028654

PMASK = {"x": 1, "y": 3, "z": 4}

BIT = {
    "x": lambda v: (v & 1) ^ ((v >> 1) & 1),
    "y": lambda v: (v >> 1) & 1,
    "z": lambda v: (v >> 2) & 1,
}


def _chunk_of(bits):
    return bits["z"] * 4 + bits["y"] * 2 + (bits["x"] ^ bits["y"])


def _gelu(y):
    return 0.5 * y * (1.0 + jnp.tanh(GELU_C * (y + 0.044715 * y * y * y)))


STREAMS = [
    (0, 384, ("z", "y", "x")),
    (384, 384, ("y", "x", "z")),
    (768, 256, ("x", "z", "y")),
]
ORDER_SIS = [2, 0, 1]

R1_COMBOS = [(1, 1), (1, 0), (0, 1), (0, 0)]
R2_COMBOS = [(1,), (0,)]
R1_BASE, R2_BASE, R3_SLOT = 0, 4, 6


def kernel(x, w_mat):
    m, k_per = x.shape
    _, n = w_mat.shape

    n_streams = len(STREAMS)

    def body(x_ref, w_ref, out_ref, p_ref, *rest):
        send_bufs = rest[0:2 * n_streams:2]
        recv_bufs = rest[1:2 * n_streams:2]
        send_sems, recv_sems = rest[2 * n_streams], rest[2 * n_streams + 1]

        my = lax.axis_index("i")
        mybit = {a: BIT[a](my) for a in "xyz"}

        barrier_sem = pltpu.get_barrier_semaphore()
        for mask in (1, 3, 4):
            pl.semaphore_signal(
                barrier_sem, inc=1,
                device_id=(my ^ mask,), device_id_type=pl.DeviceIdType.MESH,
            )
        pl.semaphore_wait(barrier_sem, 3)

        def compute_p_cols(si):
            c0, w, _ = STREAMS[si]
            p_ref[:, c0:c0 + w] = jnp.dot(
                x_ref[:, :], w_ref[:, c0:c0 + w],
                preferred_element_type=jnp.float32,
            ).astype(jnp.bfloat16)

        def send_chunk_bits(si, r, combo):
            _, _, order = STREAMS[si]
            bits = dict(mybit)
            bits[order[r]] = mybit[order[r]] ^ 1
            for j, f in enumerate(order[r + 1:]):
                bits[f] = mybit[f] ^ combo[j]
            return bits

        def rows(c):
            return pl.ds(c * M_PER, M_PER)

        def make_rdma(si, slot, a):
            sb, rb = send_bufs[si], recv_bufs[si]
            return pltpu.make_async_remote_copy(
                src_ref=sb.at[slot],
                dst_ref=rb.at[slot],
                send_sem=send_sems.at[si, slot],
                recv_sem=recv_sems.at[si, slot],
                device_id=(my ^ PMASK[a],),
                device_id_type=pl.DeviceIdType.MESH,
            )

        rdmas = {}

        def start_r1(si):
            c0, w, order = STREAMS[si]
            for k, combo in enumerate(R1_COMBOS):
                c = _chunk_of(send_chunk_bits(si, 0, combo))
                slot = R1_BASE + k
                rdma = pltpu.make_async_remote_copy(
                    src_ref=p_ref.at[rows(c), pl.ds(c0, w)],
                    dst_ref=recv_bufs[si].at[slot],
                    send_sem=send_sems.at[si, slot],
                    recv_sem=recv_sems.at[si, slot],
                    device_id=(my ^ PMASK[order[0]],),
                    device_id_type=pl.DeviceIdType.MESH,
                )
                rdmas[si, slot] = rdma
                rdma.start()

        def start_r2(si):
            c0, w, order = STREAMS[si]
            for k, combo in enumerate(R2_COMBOS):
                c = _chunk_of(send_chunk_bits(si, 1, combo))
                r1_slot = R1_BASE + (0 if combo[0] == 1 else 1)
                rdmas[si, r1_slot].wait_recv()
                slot = R2_BASE + k
                send_bufs[si][slot, :, :] = (
                    p_ref[rows(c), c0:c0 + w] + recv_bufs[si][r1_slot, :, :]
                )
                rdmas[si, slot] = make_rdma(si, slot, order[1])
                rdmas[si, slot].start()

        def start_r3(si):
            c0, w, order = STREAMS[si]
            rdmas[si, R1_BASE + 2].wait_recv()
            rdmas[si, R2_BASE + 0].wait_recv()
            c = _chunk_of(send_chunk_bits(si, 2, ()))
            send_bufs[si][R3_SLOT, :, :] = (
                p_ref[rows(c), c0:c0 + w]
                + recv_bufs[si][R1_BASE + 2, :, :]
                + recv_bufs[si][R2_BASE + 0, :, :]
            )
            rdmas[si, R3_SLOT] = make_rdma(si, R3_SLOT, order[2])
            rdmas[si, R3_SLOT].start()

        def finish(si):
            c0, w, _ = STREAMS[si]
            rdmas[si, R1_BASE + 3].wait_recv()
            rdmas[si, R2_BASE + 1].wait_recv()
            rdmas[si, R3_SLOT].wait_recv()
            acc = (
                p_ref[rows(my), c0:c0 + w].astype(jnp.float32)
                + recv_bufs[si][R1_BASE + 3, :, :].astype(jnp.float32)
                + recv_bufs[si][R2_BASE + 1, :, :].astype(jnp.float32)
                + recv_bufs[si][R3_SLOT, :, :].astype(jnp.float32)
            )
            out_ref[:, c0:c0 + w] = _gelu(acc)

        for si in ORDER_SIS:
            compute_p_cols(si)
            start_r1(si)
        for phase in (start_r2, start_r3, finish):
            for si in ORDER_SIS:
                phase(si)
        for si in ORDER_SIS:
            for slot in range(7):
                rdmas[si, slot].wait_send()

    scratch = [pltpu.VMEM((m, n), jnp.bfloat16)]
    for c0, w, order in STREAMS:
        scratch.append(pltpu.VMEM((7, M_PER, w), jnp.bfloat16))
        scratch.append(pltpu.VMEM((7, M_PER, w), jnp.bfloat16))
    scratch.append(pltpu.SemaphoreType.DMA((n_streams, 7)))
    scratch.append(pltpu.SemaphoreType.DMA((n_streams, 7)))

    return pl.pallas_call(
        body,
        out_shape=jax.ShapeDtypeStruct((M_PER, n), jnp.float32),
        in_specs=[
            pl.BlockSpec(memory_space=pltpu.VMEM),
            pl.BlockSpec(memory_space=pltpu.VMEM),
        ],
        out_specs=pl.BlockSpec(memory_space=pltpu.VMEM),
        scratch_shapes=scratch,
        compiler_params=pltpu.CompilerParams(collective_id=0),
    )(x, w_mat)


# device time: 18101 ns/iter; 1.0522x vs baseline; 1.0316x over previous
import jax
import jax.numpy as jnp
from jax import lax
from jax.experimental import pallas as pl
from jax.experimental.pallas import tpu as pltpu

N_DEV = 8
M_PER = 128
GELU_C = 0.7978845608028654

PMASK = {"x": 1, "y": 3, "z": 4}

BIT = {
    "x": lambda v: (v & 1) ^ ((v >> 1) & 1),
    "y": lambda v: (v >> 1) & 1,
    "z": lambda v: (v >> 2) & 1,
}


def _chunk_of(bits):
    return bits["z"] * 4 + bits["y"] * 2 + (bits["x"] ^ bits["y"])


def _gelu(y):
    return 0.5 * y * (1.0 + jnp.tanh(GELU_C * (y + 0.044715 * y * y * y)))


STREAMS = [
    (0, 384, ("z", "y", "x")),
    (384, 384, ("y", "x", "z")),
    (768, 256, ("x", "z", "y")),
]
ORDER_SIS = [2, 0, 1]

R1_COMBOS = [(1, 1), (1, 0), (0, 1), (0, 0)]
R2_COMBOS = [(1,), (0,)]
R1_BASE, R2_BASE, R3_SLOT = 0, 4, 6


def kernel(x, w_mat):
    m, k_per = x.shape
    _, n = w_mat.shape

    n_streams = len(STREAMS)

    def body(x_ref, w_ref, out_ref, p_ref, *rest):
        send_bufs = rest[0:2 * n_streams:2]
        recv_bufs = rest[1:2 * n_streams:2]
        send_sems, recv_sems = rest[2 * n_streams], rest[2 * n_streams + 1]

        my = lax.axis_index("i")
        mybit = {a: BIT[a](my) for a in "xyz"}

        barrier_sem = pltpu.get_barrier_semaphore()
        for mask in (1, 3, 4):
            pl.semaphore_signal(
                barrier_sem, inc=1,
                device_id=(my ^ mask,), device_id_type=pl.DeviceIdType.MESH,
            )
        pl.semaphore_wait(barrier_sem, 3)

        p_ref[:, :] = jnp.dot(
            x_ref[:, :], w_ref[:, :], preferred_element_type=jnp.float32
        ).astype(jnp.bfloat16)

        def send_chunk_bits(si, r, combo):
            _, _, order = STREAMS[si]
            bits = dict(mybit)
            bits[order[r]] = mybit[order[r]] ^ 1
            for j, f in enumerate(order[r + 1:]):
                bits[f] = mybit[f] ^ combo[j]
            return bits

        def rows(c):
            return pl.ds(c * M_PER, M_PER)

        def make_rdma(si, slot, a):
            sb, rb = send_bufs[si], recv_bufs[si]
            return pltpu.make_async_remote_copy(
                src_ref=sb.at[slot],
                dst_ref=rb.at[slot],
                send_sem=send_sems.at[si, slot],
                recv_sem=recv_sems.at[si, slot],
                device_id=(my ^ PMASK[a],),
                device_id_type=pl.DeviceIdType.MESH,
            )

        rdmas = {}

        def start_r1(si):
            c0, w, order = STREAMS[si]
            for k, combo in enumerate(R1_COMBOS):
                c = _chunk_of(send_chunk_bits(si, 0, combo))
                slot = R1_BASE + k
                send_bufs[si][slot, :, :] = p_ref[rows(c), c0:c0 + w]
                rdmas[si, slot] = make_rdma(si, slot, order[0])
                rdmas[si, slot].start()

        def start_r2(si):
            c0, w, order = STREAMS[si]
            rdmas[si, R1_BASE + 0].wait_recv()
            rdmas[si, R1_BASE + 1].wait_recv()
            for k, combo in enumerate(R2_COMBOS):
                c = _chunk_of(send_chunk_bits(si, 1, combo))
                r1_slot = R1_BASE + (0 if combo[0] == 1 else 1)
                slot = R2_BASE + k
                send_bufs[si][slot, :, :] = (
                    p_ref[rows(c), c0:c0 + w] + recv_bufs[si][r1_slot, :, :]
                )
                rdmas[si, slot] = make_rdma(si, slot, order[1])
                rdmas[si, slot].start()

        def start_r3(si):
            c0, w, order = STREAMS[si]
            rdmas[si, R1_BASE + 2].wait_recv()
            rdmas[si, R2_BASE + 0].wait_recv()
            c = _chunk_of(send_chunk_bits(si, 2, ()))
            send_bufs[si][R3_SLOT, :, :] = (
                p_ref[rows(c), c0:c0 + w]
                + recv_bufs[si][R1_BASE + 2, :, :]
                + recv_bufs[si][R2_BASE + 0, :, :]
            )
            rdmas[si, R3_SLOT] = make_rdma(si, R3_SLOT, order[2])
            rdmas[si, R3_SLOT].start()

        def finish(si):
            c0, w, _ = STREAMS[si]
            rdmas[si, R1_BASE + 3].wait_recv()
            rdmas[si, R2_BASE + 1].wait_recv()
            rdmas[si, R3_SLOT].wait_recv()
            acc = (
                p_ref[rows(my), c0:c0 + w].astype(jnp.float32)
                + recv_bufs[si][R1_BASE + 3, :, :].astype(jnp.float32)
                + recv_bufs[si][R2_BASE + 1, :, :].astype(jnp.float32)
                + recv_bufs[si][R3_SLOT, :, :].astype(jnp.float32)
            )
            out_ref[:, c0:c0 + w] = _gelu(acc)

        for si in ORDER_SIS:
            start_r1(si)
        for phase in (start_r2, start_r3, finish):
            for si in ORDER_SIS:
                phase(si)
        for si in ORDER_SIS:
            for slot in range(7):
                rdmas[si, slot].wait_send()

    scratch = [pltpu.VMEM((m, n), jnp.bfloat16)]
    for c0, w, order in STREAMS:
        scratch.append(pltpu.VMEM((7, M_PER, w), jnp.bfloat16))
        scratch.append(pltpu.VMEM((7, M_PER, w), jnp.bfloat16))
    scratch.append(pltpu.SemaphoreType.DMA((n_streams, 7)))
    scratch.append(pltpu.SemaphoreType.DMA((n_streams, 7)))

    return pl.pallas_call(
        body,
        out_shape=jax.ShapeDtypeStruct((M_PER, n), jnp.float32),
        in_specs=[
            pl.BlockSpec(memory_space=pltpu.VMEM),
            pl.BlockSpec(memory_space=pltpu.VMEM),
        ],
        out_specs=pl.BlockSpec(memory_space=pltpu.VMEM),
        scratch_shapes=scratch,
        compiler_params=pltpu.CompilerParams(collective_id=0),
    )(x, w_mat)
